# Initial kernel scaffold; baseline (speedup 1.0000x reference)
import jax
import jax.numpy as jnp
from jax.experimental import pallas as pl


def kernel(dense_features, sparse_indices, tables, Wl, bl, W1, b1, W2, b2, W3, b3, Wo, bo):
    raise NotImplementedError("write your pallas kernel here")


# speedup vs baseline: 2.2038x; 2.2038x over previous
import functools

import jax
import jax.numpy as jnp
from jax import lax
from jax.experimental import pallas as pl
from jax.experimental.pallas import tpu as pltpu
from jax.experimental.pallas import tpu_sc as plsc

B = 4096
NUM_DENSE = 13
F = 26
V = 100000
D = 32
H1, H2, H3 = 1024, 512, 256

NC, NS = 2, 16
NW = NC * NS
N_ROWS = B * F
RPW = N_ROWS // NW
CHUNK = 128
NCH = RPW // CHUNK


def _sc_gather(tab2d, idx3d):
    mesh = plsc.VectorSubcoreMesh(core_axis_name="c", subcore_axis_name="s")

    @functools.partial(
        pl.kernel,
        out_type=jax.ShapeDtypeStruct((NW, RPW, D), jnp.float32),
        mesh=mesh,
        scratch_types=[
            pltpu.VMEM((NCH, CHUNK), jnp.int32),
            pltpu.VMEM((RPW, D), jnp.float32),
            pltpu.SemaphoreType.DMA,
        ],
        compiler_params=pltpu.CompilerParams(use_tc_tiling_on_sc=False),
    )
    def k(tab_hbm, idx_hbm, out_hbm, idx_v, rows_v, sem):
        wid = lax.axis_index("s") * NC + lax.axis_index("c")
        pltpu.sync_copy(idx_hbm.at[wid], idx_v)
        copies = [
            pltpu.async_copy(
                tab_hbm.at[idx_v.at[j]],
                rows_v.at[pl.ds(j * CHUNK, CHUNK)],
                sem,
            )
            for j in range(NCH)
        ]
        for cp in copies:
            cp.wait()
        pltpu.sync_copy(rows_v, out_hbm.at[wid])

    return k(tab2d, idx3d)


def _mlp_body(emb, dense, wl, bl, w1, b1, w2, b2, w3, b3, woh, wot, bo, out):
    f32 = jnp.float32
    wide = jnp.dot(dense[:], wl[:], preferred_element_type=f32) + bl[0, 0]
    h = jnp.dot(emb[:], w1[:], preferred_element_type=f32) + b1[:]
    h = jnp.maximum(h, 0.0)
    h = jnp.dot(h, w2[:], preferred_element_type=f32) + b2[:]
    h = jnp.maximum(h, 0.0)
    h = jnp.dot(h, w3[:], preferred_element_type=f32) + b3[:]
    h = jnp.maximum(h, 0.0)
    logit = wide * woh[0, 0] + jnp.dot(h, wot[:], preferred_element_type=f32)
    out[:] = jax.nn.sigmoid(logit + bo[0, 0])


def _mlp(emb, dense, wl, bl, w1, b1, w2, b2, w3, b3, woh, wot, bo):
    BB = 512
    grid = (B // BB,)

    def batch_block(shape):
        return pl.BlockSpec((BB, shape), lambda i: (i, 0))

    def full(a, b):
        return pl.BlockSpec((a, b), lambda i: (0, 0))

    return pl.pallas_call(
        _mlp_body,
        grid=grid,
        in_specs=[
            batch_block(F * D),
            batch_block(NUM_DENSE),
            full(NUM_DENSE, 1),
            full(1, 1),
            full(F * D, H1),
            full(1, H1),
            full(H1, H2),
            full(1, H2),
            full(H2, H3),
            full(1, H3),
            full(1, 1),
            full(H3, 1),
            full(1, 1),
        ],
        out_specs=batch_block(1),
        out_shape=jax.ShapeDtypeStruct((B, 1), jnp.float32),
    )(emb, dense, wl, bl, w1, b1, w2, b2, w3, b3, woh, wot, bo)


def kernel(dense_features, sparse_indices, tables, Wl, bl, W1, b1, W2, b2, W3, b3, Wo, bo):
    idx = sparse_indices.astype(jnp.int32)
    flat_idx = (idx + (jnp.arange(F, dtype=jnp.int32) * V)[None, :]).reshape(NW, NCH, CHUNK)
    tab2d = tables.reshape(F * V, D)
    rows = _sc_gather(tab2d, flat_idx)
    emb = rows.reshape(B, F * D)
    out = _mlp(
        emb,
        dense_features,
        Wl,
        bl.reshape(1, 1),
        W1,
        b1.reshape(1, H1),
        W2,
        b2.reshape(1, H2),
        W3,
        b3.reshape(1, H3),
        Wo[0:1, :],
        Wo[1:, :],
        bo.reshape(1, 1),
    )
    return out


# speedup vs baseline: 4.2922x; 1.9477x over previous
import functools

import jax
import jax.numpy as jnp
from jax import lax
from jax.experimental import pallas as pl
from jax.experimental.pallas import tpu as pltpu
from jax.experimental.pallas import tpu_sc as plsc

B = 4096
NUM_DENSE = 13
F = 26
V = 100000
D = 32
H1, H2, H3 = 1024, 512, 256

Q = 25088
BG = 1792
NQB = Q // BG

NC, NS = 2, 16
NW = NC * NS
N_ROWS = B * F
RPW = N_ROWS // NW
CHUNK = 256
NCHK = RPW // CHUNK


def _relayout_body(t0, t1, t2, t3, out):
    tstack = jnp.concatenate([t0[0], t1[0], t2[0], t3[0]], axis=0)
    out[0] = jnp.transpose(tstack)


def _relayout(tabT):
    def in_spec(q):
        return pl.BlockSpec((1, D, BG), lambda f, g, q=q: (f, 0, NQB * q + g))

    return pl.pallas_call(
        _relayout_body,
        grid=(F, NQB),
        in_specs=[in_spec(0), in_spec(1), in_spec(2), in_spec(3)],
        out_specs=pl.BlockSpec((1, BG, 128), lambda f, g: (f, g, 0)),
        out_shape=jax.ShapeDtypeStruct((F, Q, 128), jnp.float32),
    )(tabT, tabT, tabT, tabT)


def _sc_gather(g2, idx3d):
    mesh = plsc.VectorSubcoreMesh(core_axis_name="c", subcore_axis_name="s")

    @functools.partial(
        pl.kernel,
        out_type=jax.ShapeDtypeStruct((NW, RPW, 128), jnp.float32),
        mesh=mesh,
        scratch_types=[
            pltpu.VMEM((32, 128), jnp.int32),
            pltpu.VMEM((2, CHUNK, 128), jnp.float32),
            pltpu.SemaphoreType.DMA,
            pltpu.SemaphoreType.DMA,
        ],
        compiler_params=pltpu.CompilerParams(use_tc_tiling_on_sc=True),
    )
    def k(g_hbm, idx_hbm, out_hbm, idx_v, rows_v, gsem, wsem):
        wid = lax.axis_index("s") * NC + lax.axis_index("c")
        pltpu.sync_copy(idx_hbm.at[wid], idx_v)

        def fire(c):
            buf = c % 2
            return [
                pltpu.async_copy(
                    g_hbm.at[idx_v.at[2 * c + j]],
                    rows_v.at[buf, pl.ds(j * 128, 128)],
                    gsem,
                )
                for j in range(2)
            ]

        writes = []
        gathers = fire(0)
        for c in range(NCHK):
            for cp in gathers:
                cp.wait()
            writes.append(
                pltpu.async_copy(
                    rows_v.at[c % 2],
                    out_hbm.at[wid, pl.ds(c * CHUNK, CHUNK)],
                    wsem,
                )
            )
            if c + 1 < NCHK:
                if len(writes) >= 2:
                    writes.pop(0).wait()
                gathers = fire(c + 1)
        for w in writes:
            w.wait()

    return k(g2, idx3d)


def _mlp_body(xw, qoff, dense, wl, bl, w1, b1, w2, b2, w3, b3, woh, wot, bo,
              out, x_s):
    f32 = jnp.float32
    xw3 = xw[:]
    qf = qoff[:].reshape(qoff.shape[0], F, 1)
    x3 = jnp.where(qf == 0, xw3[:, :, 0:D], 0.0)
    for q in range(1, 4):
        x3 = x3 + jnp.where(qf == q, xw3[:, :, q * D:(q + 1) * D], 0.0)
    for f in range(F):
        x_s[:, f * D:(f + 1) * D] = x3[:, f, :]
    x = x_s[:]
    wide = jnp.dot(dense[:], wl[:], preferred_element_type=f32) + bl[0, 0]
    h = jnp.dot(x, w1[:], preferred_element_type=f32) + b1[:]
    h = jnp.maximum(h, 0.0)
    h = jnp.dot(h, w2[:], preferred_element_type=f32) + b2[:]
    h = jnp.maximum(h, 0.0)
    h = jnp.dot(h, w3[:], preferred_element_type=f32) + b3[:]
    h = jnp.maximum(h, 0.0)
    logit = wide * woh[0, 0] + jnp.dot(h, wot[:], preferred_element_type=f32)
    out[:] = jax.nn.sigmoid(logit + bo[0, 0])


def _mlp(xw, qoff, dense, wl, bl, w1, b1, w2, b2, w3, b3, woh, wot, bo):
    BB = 512
    grid = (B // BB,)

    def full(a, b):
        return pl.BlockSpec((a, b), lambda i: (0, 0))

    return pl.pallas_call(
        _mlp_body,
        grid=grid,
        in_specs=[
            pl.BlockSpec((BB, F, 128), lambda i: (i, 0, 0)),
            pl.BlockSpec((BB, F), lambda i: (i, 0)),
            pl.BlockSpec((BB, NUM_DENSE), lambda i: (i, 0)),
            full(NUM_DENSE, 1),
            full(1, 1),
            full(F * D, H1),
            full(1, H1),
            full(H1, H2),
            full(1, H2),
            full(H2, H3),
            full(1, H3),
            full(1, 1),
            full(H3, 1),
            full(1, 1),
        ],
        out_specs=pl.BlockSpec((BB, 1), lambda i: (i, 0)),
        out_shape=jax.ShapeDtypeStruct((B, 1), jnp.float32),
        scratch_shapes=[pltpu.VMEM((BB, F * D), jnp.float32)],
    )(xw, qoff, dense, wl, bl, w1, b1, w2, b2, w3, b3, woh, wot, bo)


def kernel(dense_features, sparse_indices, tables, Wl, bl, W1, b1, W2, b2, W3, b3, Wo, bo):
    idx = sparse_indices.astype(jnp.int32)
    tabT = jnp.transpose(tables, (0, 2, 1))
    g = _relayout(tabT)
    g2 = g.reshape(F * Q, 128)

    qsel = idx // Q
    m = idx - qsel * Q + (jnp.arange(F, dtype=jnp.int32) * Q)[None, :]
    m3 = m.reshape(NW, RPW // 128, 128)
    m3 = jnp.pad(m3, ((0, 0), (0, 32 - RPW // 128), (0, 0)))

    rows = _sc_gather(g2, m3)
    xw = rows.reshape(B, F, 128)
    out = _mlp(
        xw,
        qsel,
        dense_features,
        Wl,
        bl.reshape(1, 1),
        W1,
        b1.reshape(1, H1),
        W2,
        b2.reshape(1, H2),
        W3,
        b3.reshape(1, H3),
        Wo[0:1, :],
        Wo[1:, :],
        bo.reshape(1, 1),
    )
    return out


# speedup vs baseline: 5.1224x; 1.1934x over previous
import functools

import jax
import jax.numpy as jnp
from jax import lax
from jax.experimental import pallas as pl
from jax.experimental.pallas import tpu as pltpu
from jax.experimental.pallas import tpu_sc as plsc

B = 4096
NUM_DENSE = 13
F = 26
V = 100000
D = 32
H1, H2, H3 = 1024, 512, 256

Q = 25088
BG = 1792
NQB = Q // BG

NC, NS = 2, 16
NW = NC * NS
N_ROWS = B * F
RPW = N_ROWS // NW
CHUNK = 256
NCHK = RPW // CHUNK


def _relayout_body(t0, t1, t2, t3, out):
    tstack = jnp.concatenate([t0[0], t1[0], t2[0], t3[0]], axis=0)
    out[0] = jnp.transpose(tstack)


def _relayout(tabT):
    def in_spec(q):
        return pl.BlockSpec((1, D, BG), lambda f, g, q=q: (f, 0, NQB * q + g))

    return pl.pallas_call(
        _relayout_body,
        grid=(F, NQB),
        in_specs=[in_spec(0), in_spec(1), in_spec(2), in_spec(3)],
        out_specs=pl.BlockSpec((1, BG, 128), lambda f, g: (f, g, 0)),
        out_shape=jax.ShapeDtypeStruct((F, Q, 128), jnp.float32),
    )(tabT, tabT, tabT, tabT)


def _sc_gather(g2, idx3d):
    mesh = plsc.VectorSubcoreMesh(core_axis_name="c", subcore_axis_name="s")

    @functools.partial(
        pl.kernel,
        out_type=jax.ShapeDtypeStruct((N_ROWS, 128), jnp.float32),
        mesh=mesh,
        scratch_types=[
            pltpu.VMEM((32, 128), jnp.int32),
            pltpu.VMEM((2, CHUNK, 128), jnp.float32),
            pltpu.SemaphoreType.DMA,
            pltpu.SemaphoreType.DMA,
        ],
        compiler_params=pltpu.CompilerParams(use_tc_tiling_on_sc=True),
    )
    def k(g_hbm, idx_hbm, out_hbm, idx_v, rows_v, gsem, wsem):
        wid = lax.axis_index("s") * NC + lax.axis_index("c")
        pltpu.sync_copy(idx_hbm.at[wid], idx_v)

        def fire(c):
            buf = c % 2
            return [
                pltpu.async_copy(
                    g_hbm.at[idx_v.at[2 * c + j]],
                    rows_v.at[buf, pl.ds(j * 128, 128)],
                    gsem,
                )
                for j in range(2)
            ]

        writes = []
        gathers = fire(0)
        base = wid * RPW
        for c in range(NCHK):
            for cp in gathers:
                cp.wait()
            writes.append(
                pltpu.async_copy(
                    rows_v.at[c % 2],
                    out_hbm.at[pl.ds(base + c * CHUNK, CHUNK)],
                    wsem,
                )
            )
            if c + 1 < NCHK:
                if len(writes) >= 2:
                    writes.pop(0).wait()
                gathers = fire(c + 1)
        for w in writes:
            w.wait()

    return k(g2, idx3d)


def _mlp_body(xw, qoff, dense, wl, bl, w1r, b1, w2, b2, w3, b3, woh, wot, bo,
              out):
    f32 = jnp.float32
    lane_q = lax.broadcasted_iota(jnp.int32, (1, 128), 1) // D
    h = jnp.broadcast_to(b1[:], (qoff.shape[0], H1))
    for f in range(F):
        qv = qoff[:, f:f + 1]
        mask = (lane_q == qv).astype(f32)
        xm = xw[f] * mask
        h = h + jnp.dot(xm, w1r[f], preferred_element_type=f32)
    h = jnp.maximum(h, 0.0)
    wide = jnp.dot(dense[:], wl[:], preferred_element_type=f32) + bl[0, 0]
    h = jnp.dot(h, w2[:], preferred_element_type=f32) + b2[:]
    h = jnp.maximum(h, 0.0)
    h = jnp.dot(h, w3[:], preferred_element_type=f32) + b3[:]
    h = jnp.maximum(h, 0.0)
    logit = wide * woh[0, 0] + jnp.dot(h, wot[:], preferred_element_type=f32)
    out[:] = jax.nn.sigmoid(logit + bo[0, 0])


def _mlp(xw3, qoff, dense, wl, bl, w1r, b1, w2, b2, w3, b3, woh, wot, bo):
    BB = 512
    grid = (B // BB,)

    def full(*dims):
        return pl.BlockSpec(dims, lambda i, dims=dims: tuple(0 for _ in dims))

    return pl.pallas_call(
        _mlp_body,
        grid=grid,
        in_specs=[
            pl.BlockSpec((F, BB, 128), lambda i: (0, i, 0)),
            pl.BlockSpec((BB, F), lambda i: (i, 0)),
            pl.BlockSpec((BB, NUM_DENSE), lambda i: (i, 0)),
            full(NUM_DENSE, 1),
            full(1, 1),
            full(F, 128, H1),
            full(1, H1),
            full(H1, H2),
            full(1, H2),
            full(H2, H3),
            full(1, H3),
            full(1, 1),
            full(H3, 1),
            full(1, 1),
        ],
        out_specs=pl.BlockSpec((BB, 1), lambda i: (i, 0)),
        out_shape=jax.ShapeDtypeStruct((B, 1), jnp.float32),
    )(xw3, qoff, dense, wl, bl, w1r, b1, w2, b2, w3, b3, woh, wot, bo)


def kernel(dense_features, sparse_indices, tables, Wl, bl, W1, b1, W2, b2, W3, b3, Wo, bo):
    idx = sparse_indices.astype(jnp.int32)
    tabT = jnp.transpose(tables, (0, 2, 1))
    g = _relayout(tabT)
    g2 = g.reshape(F * Q, 128)

    qsel = idx // Q
    m = idx - qsel * Q + (jnp.arange(F, dtype=jnp.int32) * Q)[None, :]
    m3 = m.T.reshape(NW, RPW // 128, 128)
    m3 = jnp.pad(m3, ((0, 0), (0, 32 - RPW // 128), (0, 0)))

    rows = _sc_gather(g2, m3)
    xw3 = rows.reshape(F, B, 128)
    w1r = jnp.broadcast_to(W1.reshape(F, 1, D, H1), (F, 4, D, H1))
    w1r = w1r.reshape(F, 128, H1)
    out = _mlp(
        xw3,
        qsel,
        dense_features,
        Wl,
        bl.reshape(1, 1),
        w1r,
        b1.reshape(1, H1),
        W2,
        b2.reshape(1, H2),
        W3,
        b3.reshape(1, H3),
        Wo[0:1, :],
        Wo[1:, :],
        bo.reshape(1, 1),
    )
    return out


# speedup vs baseline: 7.2750x; 1.4202x over previous
import functools

import jax
import jax.numpy as jnp
from jax import lax
from jax.experimental import pallas as pl
from jax.experimental.pallas import tpu as pltpu
from jax.experimental.pallas import tpu_sc as plsc

B = 4096
NUM_DENSE = 13
F = 26
V = 100000
D = 32
H1, H2, H3 = 1024, 512, 256

Q = 12544
BG = 1792
NQB = Q // BG

NC, NS = 2, 16
NW = NC * NS
N_ROWS = B * F
RPW = N_ROWS // NW
CHUNK = 256
NCHK = RPW // CHUNK


def _pack_pair(lo, hi):
    lo16 = lax.bitcast_convert_type(lo.astype(jnp.bfloat16), jnp.uint16)
    hi16 = lax.bitcast_convert_type(hi.astype(jnp.bfloat16), jnp.uint16)
    word = lo16.astype(jnp.uint32) | (hi16.astype(jnp.uint32) << 16)
    return lax.bitcast_convert_type(word, jnp.float32)


def _relayout_body(t0, t1, t2, t3, t4, t5, t6, t7, out):
    ts = (t0, t1, t2, t3, t4, t5, t6, t7)
    packed = [_pack_pair(ts[2 * a][0], ts[2 * a + 1][0]) for a in range(4)]
    tstack = jnp.concatenate(packed, axis=0)
    out[0] = jnp.transpose(tstack)


def _relayout(tabT):
    def in_spec(s):
        return pl.BlockSpec((1, D, BG), lambda f, g, s=s: (f, 0, NQB * s + g))

    return pl.pallas_call(
        _relayout_body,
        grid=(F, NQB),
        in_specs=[in_spec(s) for s in range(8)],
        out_specs=pl.BlockSpec((1, BG, 128), lambda f, g: (f, g, 0)),
        out_shape=jax.ShapeDtypeStruct((F, Q, 128), jnp.float32),
    )(*([tabT] * 8))


def _sc_gather(g2, idx3d):
    mesh = plsc.VectorSubcoreMesh(core_axis_name="c", subcore_axis_name="s")

    @functools.partial(
        pl.kernel,
        out_type=jax.ShapeDtypeStruct((N_ROWS, 128), jnp.float32),
        mesh=mesh,
        scratch_types=[
            pltpu.VMEM((32, 128), jnp.int32),
            pltpu.VMEM((2, CHUNK, 128), jnp.float32),
            pltpu.SemaphoreType.DMA,
            pltpu.SemaphoreType.DMA,
        ],
        compiler_params=pltpu.CompilerParams(use_tc_tiling_on_sc=True),
    )
    def k(g_hbm, idx_hbm, out_hbm, idx_v, rows_v, gsem, wsem):
        wid = lax.axis_index("s") * NC + lax.axis_index("c")
        pltpu.sync_copy(idx_hbm.at[wid], idx_v)

        def fire(c):
            buf = c % 2
            return [
                pltpu.async_copy(
                    g_hbm.at[idx_v.at[2 * c + j]],
                    rows_v.at[buf, pl.ds(j * 128, 128)],
                    gsem,
                )
                for j in range(2)
            ]

        writes = []
        gathers = fire(0)
        base = wid * RPW
        for c in range(NCHK):
            for cp in gathers:
                cp.wait()
            writes.append(
                pltpu.async_copy(
                    rows_v.at[c % 2],
                    out_hbm.at[pl.ds(base + c * CHUNK, CHUNK)],
                    wsem,
                )
            )
            if c + 1 < NCHK:
                if len(writes) >= 2:
                    writes.pop(0).wait()
                gathers = fire(c + 1)
        for w in writes:
            w.wait()

    return k(g2, idx3d)


def _mlp_body(xw, qoff, dense, wl, bl, w1r, b1, w2, b2, w3, b3, woh, wot, bo,
              out):
    f32 = jnp.float32
    lane_q = lax.broadcasted_iota(jnp.int32, (1, 128), 1) // D
    himask = jnp.uint32(0xFFFF0000)
    h = jnp.broadcast_to(b1[:], (qoff.shape[0], H1))
    for f in range(F):
        qv = qoff[:, f:f + 1]
        mask = (lane_q == (qv >> 1)).astype(f32)
        shift = ((1 - (qv & 1)) << 4).astype(jnp.uint32)
        xu = lax.bitcast_convert_type(xw[f], jnp.uint32)
        xv = lax.bitcast_convert_type((xu << shift) & himask, f32)
        xm = xv * mask
        h = h + jnp.dot(xm, w1r[f], preferred_element_type=f32)
    h = jnp.maximum(h, 0.0)
    wide = jnp.dot(dense[:], wl[:], preferred_element_type=f32) + bl[0, 0]
    h = jnp.dot(h, w2[:], preferred_element_type=f32) + b2[:]
    h = jnp.maximum(h, 0.0)
    h = jnp.dot(h, w3[:], preferred_element_type=f32) + b3[:]
    h = jnp.maximum(h, 0.0)
    logit = wide * woh[0, 0] + jnp.dot(h, wot[:], preferred_element_type=f32)
    out[:] = jax.nn.sigmoid(logit + bo[0, 0])


def _mlp(xw3, qoff, dense, wl, bl, w1r, b1, w2, b2, w3, b3, woh, wot, bo):
    BB = 512
    grid = (B // BB,)

    def full(*dims):
        return pl.BlockSpec(dims, lambda i, dims=dims: tuple(0 for _ in dims))

    return pl.pallas_call(
        _mlp_body,
        grid=grid,
        in_specs=[
            pl.BlockSpec((F, BB, 128), lambda i: (0, i, 0)),
            pl.BlockSpec((BB, F), lambda i: (i, 0)),
            pl.BlockSpec((BB, NUM_DENSE), lambda i: (i, 0)),
            full(NUM_DENSE, 1),
            full(1, 1),
            full(F, 128, H1),
            full(1, H1),
            full(H1, H2),
            full(1, H2),
            full(H2, H3),
            full(1, H3),
            full(1, 1),
            full(H3, 1),
            full(1, 1),
        ],
        out_specs=pl.BlockSpec((BB, 1), lambda i: (i, 0)),
        out_shape=jax.ShapeDtypeStruct((B, 1), jnp.float32),
    )(xw3, qoff, dense, wl, bl, w1r, b1, w2, b2, w3, b3, woh, wot, bo)


def kernel(dense_features, sparse_indices, tables, Wl, bl, W1, b1, W2, b2, W3, b3, Wo, bo):
    idx = sparse_indices.astype(jnp.int32)
    tabT = jnp.transpose(tables, (0, 2, 1))
    g = _relayout(tabT)
    g2 = g.reshape(F * Q, 128)

    qsel = idx // Q
    m = idx - qsel * Q + (jnp.arange(F, dtype=jnp.int32) * Q)[None, :]
    m3 = m.T.reshape(NW, RPW // 128, 128)
    m3 = jnp.pad(m3, ((0, 0), (0, 32 - RPW // 128), (0, 0)))

    rows = _sc_gather(g2, m3)
    xw3 = rows.reshape(F, B, 128)
    w1r = jnp.broadcast_to(W1.reshape(F, 1, D, H1), (F, 4, D, H1))
    w1r = w1r.reshape(F, 128, H1)
    out = _mlp(
        xw3,
        qsel,
        dense_features,
        Wl,
        bl.reshape(1, 1),
        w1r,
        b1.reshape(1, H1),
        W2,
        b2.reshape(1, H2),
        W3,
        b3.reshape(1, H3),
        Wo[0:1, :],
        Wo[1:, :],
        bo.reshape(1, 1),
    )
    return out


# speedup vs baseline: 7.2848x; 1.0013x over previous
import functools

import jax
import jax.numpy as jnp
from jax import lax
from jax.experimental import pallas as pl
from jax.experimental.pallas import tpu as pltpu
from jax.experimental.pallas import tpu_sc as plsc

B = 4096
NUM_DENSE = 13
F = 26
V = 100000
D = 32
H1, H2, H3 = 1024, 512, 256

Q = 12544
BG = 1792
NQB = Q // BG

NC, NS = 2, 16
NW = NC * NS
FH = F // 2
N_ROWS_H = B * FH
RPW = N_ROWS_H // NW
CHUNK = 128
NCHK = RPW // CHUNK


def _pack_pair(lo, hi):
    lo16 = lax.bitcast_convert_type(lo.astype(jnp.bfloat16), jnp.uint16)
    hi16 = lax.bitcast_convert_type(hi.astype(jnp.bfloat16), jnp.uint16)
    word = lo16.astype(jnp.uint32) | (hi16.astype(jnp.uint32) << 16)
    return lax.bitcast_convert_type(word, jnp.float32)


def _relayout_body(t0, t1, t2, t3, t4, t5, t6, t7, out):
    ts = (t0, t1, t2, t3, t4, t5, t6, t7)
    packed = [_pack_pair(ts[2 * a][0], ts[2 * a + 1][0]) for a in range(4)]
    tstack = jnp.concatenate(packed, axis=0)
    out[0] = jnp.transpose(tstack)


def _relayout(tabT, f0):
    def in_spec(s):
        return pl.BlockSpec(
            (1, D, BG), lambda f, g, s=s: (f0 + f, 0, NQB * s + g))

    return pl.pallas_call(
        _relayout_body,
        grid=(FH, NQB),
        in_specs=[in_spec(s) for s in range(8)],
        out_specs=pl.BlockSpec((1, BG, 128), lambda f, g: (f, g, 0)),
        out_shape=jax.ShapeDtypeStruct((FH, Q, 128), jnp.float32),
    )(*([tabT] * 8))


def _sc_gather(g2, idx3d):
    mesh = plsc.VectorSubcoreMesh(core_axis_name="c", subcore_axis_name="s")

    @functools.partial(
        pl.kernel,
        out_type=jax.ShapeDtypeStruct((N_ROWS_H, 128), jnp.float32),
        mesh=mesh,
        scratch_types=[
            pltpu.VMEM((16, 128), jnp.int32),
            pltpu.VMEM((2, CHUNK, 128), jnp.float32),
            pltpu.SemaphoreType.DMA,
            pltpu.SemaphoreType.DMA,
        ],
        compiler_params=pltpu.CompilerParams(use_tc_tiling_on_sc=True),
    )
    def k(g_hbm, idx_hbm, out_hbm, idx_v, rows_v, gsem, wsem):
        wid = lax.axis_index("s") * NC + lax.axis_index("c")
        pltpu.sync_copy(idx_hbm.at[wid], idx_v)

        def fire(c):
            return pltpu.async_copy(
                g_hbm.at[idx_v.at[c]], rows_v.at[c % 2], gsem)

        writes = []
        gather = fire(0)
        base = wid * RPW
        for c in range(NCHK):
            gather.wait()
            writes.append(
                pltpu.async_copy(
                    rows_v.at[c % 2],
                    out_hbm.at[pl.ds(base + c * CHUNK, CHUNK)],
                    wsem,
                )
            )
            if c + 1 < NCHK:
                if len(writes) >= 2:
                    writes.pop(0).wait()
                gather = fire(c + 1)
        for w in writes:
            w.wait()

    return k(g2, idx3d)


def _mlp_body(xwa, xwb, qoff, dense, wl, bl, w1r, b1, w2, b2, w3, b3, woh,
              wot, bo, out):
    f32 = jnp.float32
    lane_q = lax.broadcasted_iota(jnp.int32, (1, 128), 1) // D
    himask = jnp.uint32(0xFFFF0000)
    h = jnp.broadcast_to(b1[:], (qoff.shape[0], H1))
    for f in range(F):
        xw = xwa if f < FH else xwb
        qv = qoff[:, f:f + 1]
        mask = (lane_q == (qv >> 1)).astype(f32)
        shift = ((1 - (qv & 1)) << 4).astype(jnp.uint32)
        xu = lax.bitcast_convert_type(xw[f % FH], jnp.uint32)
        xv = lax.bitcast_convert_type((xu << shift) & himask, f32)
        xm = xv * mask
        h = h + jnp.dot(xm, w1r[f], preferred_element_type=f32)
    h = jnp.maximum(h, 0.0)
    wide = jnp.dot(dense[:], wl[:], preferred_element_type=f32) + bl[0, 0]
    h = jnp.dot(h, w2[:], preferred_element_type=f32) + b2[:]
    h = jnp.maximum(h, 0.0)
    h = jnp.dot(h, w3[:], preferred_element_type=f32) + b3[:]
    h = jnp.maximum(h, 0.0)
    logit = wide * woh[0, 0] + jnp.dot(h, wot[:], preferred_element_type=f32)
    out[:] = jax.nn.sigmoid(logit + bo[0, 0])


def _mlp(xw3a, xw3b, qoff, dense, wl, bl, w1r, b1, w2, b2, w3, b3, woh, wot,
         bo):
    BB = 512
    grid = (B // BB,)

    def full(*dims):
        return pl.BlockSpec(dims, lambda i, dims=dims: tuple(0 for _ in dims))

    return pl.pallas_call(
        _mlp_body,
        grid=grid,
        in_specs=[
            pl.BlockSpec((FH, BB, 128), lambda i: (0, i, 0)),
            pl.BlockSpec((FH, BB, 128), lambda i: (0, i, 0)),
            pl.BlockSpec((BB, F), lambda i: (i, 0)),
            pl.BlockSpec((BB, NUM_DENSE), lambda i: (i, 0)),
            full(NUM_DENSE, 1),
            full(1, 1),
            full(F, 128, H1),
            full(1, H1),
            full(H1, H2),
            full(1, H2),
            full(H2, H3),
            full(1, H3),
            full(1, 1),
            full(H3, 1),
            full(1, 1),
        ],
        out_specs=pl.BlockSpec((BB, 1), lambda i: (i, 0)),
        out_shape=jax.ShapeDtypeStruct((B, 1), jnp.float32),
    )(xw3a, xw3b, qoff, dense, wl, bl, w1r, b1, w2, b2, w3, b3, woh, wot, bo)


def kernel(dense_features, sparse_indices, tables, Wl, bl, W1, b1, W2, b2, W3, b3, Wo, bo):
    idx = sparse_indices.astype(jnp.int32)
    tabT = jnp.transpose(tables, (0, 2, 1))

    qsel = idx // Q
    vr = idx - qsel * Q
    floc = (jnp.arange(FH, dtype=jnp.int32) * Q)[None, :]

    def half_idx(h):
        mh = vr[:, h * FH:(h + 1) * FH] + floc
        m3 = mh.T.reshape(NW, RPW // 128, 128)
        return jnp.pad(m3, ((0, 0), (0, 16 - RPW // 128), (0, 0)))

    g_a = _relayout(tabT, 0)
    rows_a = _sc_gather(g_a.reshape(FH * Q, 128), half_idx(0))
    g_b = _relayout(tabT, FH)
    rows_b = _sc_gather(g_b.reshape(FH * Q, 128), half_idx(1))
    xw3a = rows_a.reshape(FH, B, 128)
    xw3b = rows_b.reshape(FH, B, 128)
    w1r = jnp.broadcast_to(W1.reshape(F, 1, D, H1), (F, 4, D, H1))
    w1r = w1r.reshape(F, 128, H1)
    out = _mlp(
        xw3a,
        xw3b,
        qsel,
        dense_features,
        Wl,
        bl.reshape(1, 1),
        w1r,
        b1.reshape(1, H1),
        W2,
        b2.reshape(1, H2),
        W3,
        b3.reshape(1, H3),
        Wo[0:1, :],
        Wo[1:, :],
        bo.reshape(1, 1),
    )
    return out


# speedup vs baseline: 8.5581x; 1.1748x over previous
import functools

import jax
import jax.numpy as jnp
from jax import lax
from jax.experimental import pallas as pl
from jax.experimental.pallas import tpu as pltpu
from jax.experimental.pallas import tpu_sc as plsc

B = 4096
NUM_DENSE = 13
F = 26
V = 100000
D = 32
H1, H2, H3 = 1024, 512, 256

Q = 12544
BG = 6272
NQB = Q // BG

NC, NS = 2, 16
NW = NC * NS
FH = F // 2
N_ROWS_H = B * FH
RPW = N_ROWS_H // NW
CHUNK = 128
NCHK = RPW // CHUNK


def _pack_pair(lo, hi):
    lo16 = lax.bitcast_convert_type(lo.astype(jnp.bfloat16), jnp.uint16)
    hi16 = lax.bitcast_convert_type(hi.astype(jnp.bfloat16), jnp.uint16)
    word = lo16.astype(jnp.uint32) | (hi16.astype(jnp.uint32) << 16)
    return lax.bitcast_convert_type(word, jnp.float32)


def _relayout_body(t0, t1, t2, t3, t4, t5, t6, t7, out):
    ts = (t0, t1, t2, t3, t4, t5, t6, t7)
    packed = [_pack_pair(ts[2 * a][0], ts[2 * a + 1][0]) for a in range(4)]
    tstack = jnp.concatenate(packed, axis=0)
    out[0] = jnp.transpose(tstack)


def _relayout(tabT, f0):
    def in_spec(s):
        return pl.BlockSpec(
            (1, D, BG), lambda f, g, s=s: (f0 + f, 0, NQB * s + g))

    return pl.pallas_call(
        _relayout_body,
        grid=(FH, NQB),
        in_specs=[in_spec(s) for s in range(8)],
        out_specs=pl.BlockSpec((1, BG, 128), lambda f, g: (f, g, 0)),
        out_shape=jax.ShapeDtypeStruct((FH, Q, 128), jnp.float32),
    )(*([tabT] * 8))


def _sc_gather(g2, idx3d):
    mesh = plsc.VectorSubcoreMesh(core_axis_name="c", subcore_axis_name="s")

    @functools.partial(
        pl.kernel,
        out_type=jax.ShapeDtypeStruct((N_ROWS_H, 128), jnp.float32),
        mesh=mesh,
        scratch_types=[
            pltpu.VMEM((16, 128), jnp.int32),
            pltpu.VMEM((2, CHUNK, 128), jnp.float32),
            pltpu.SemaphoreType.DMA,
            pltpu.SemaphoreType.DMA,
        ],
        compiler_params=pltpu.CompilerParams(use_tc_tiling_on_sc=True),
    )
    def k(g_hbm, idx_hbm, out_hbm, idx_v, rows_v, gsem, wsem):
        wid = lax.axis_index("s") * NC + lax.axis_index("c")
        pltpu.sync_copy(idx_hbm.at[wid], idx_v)

        def fire(c):
            return pltpu.async_copy(
                g_hbm.at[idx_v.at[c]], rows_v.at[c % 2], gsem)

        writes = []
        gather = fire(0)
        base = wid * RPW
        for c in range(NCHK):
            gather.wait()
            writes.append(
                pltpu.async_copy(
                    rows_v.at[c % 2],
                    out_hbm.at[pl.ds(base + c * CHUNK, CHUNK)],
                    wsem,
                )
            )
            if c + 1 < NCHK:
                if len(writes) >= 2:
                    writes.pop(0).wait()
                gather = fire(c + 1)
        for w in writes:
            w.wait()

    return k(g2, idx3d)


def _mlp_body(xwa, xwb, qoff, dense, wl, bl, w1r, b1, w2, b2, w3, b3, woh,
              wot, bo, out):
    f32 = jnp.float32
    lane_q = lax.broadcasted_iota(jnp.int32, (1, 128), 1) // D
    himask = jnp.uint32(0xFFFF0000)
    h = jnp.broadcast_to(b1[:], (qoff.shape[0], H1))
    for f in range(F):
        xw = xwa if f < FH else xwb
        qv = qoff[:, f:f + 1]
        mask = (lane_q == (qv >> 1)).astype(f32)
        shift = ((1 - (qv & 1)) << 4).astype(jnp.uint32)
        xu = lax.bitcast_convert_type(xw[f % FH], jnp.uint32)
        xv = lax.bitcast_convert_type((xu << shift) & himask, f32)
        xm = xv * mask
        h = h + jnp.dot(xm, w1r[f], preferred_element_type=f32)
    h = jnp.maximum(h, 0.0)
    wide = jnp.dot(dense[:], wl[:], preferred_element_type=f32) + bl[0, 0]
    h = jnp.dot(h, w2[:], preferred_element_type=f32) + b2[:]
    h = jnp.maximum(h, 0.0)
    h = jnp.dot(h, w3[:], preferred_element_type=f32) + b3[:]
    h = jnp.maximum(h, 0.0)
    logit = wide * woh[0, 0] + jnp.dot(h, wot[:], preferred_element_type=f32)
    out[:] = jax.nn.sigmoid(logit + bo[0, 0])


def _mlp(xw3a, xw3b, qoff, dense, wl, bl, w1r, b1, w2, b2, w3, b3, woh, wot,
         bo):
    BB = 512
    grid = (B // BB,)

    def full(*dims):
        return pl.BlockSpec(dims, lambda i, dims=dims: tuple(0 for _ in dims))

    return pl.pallas_call(
        _mlp_body,
        grid=grid,
        in_specs=[
            pl.BlockSpec((FH, BB, 128), lambda i: (0, i, 0)),
            pl.BlockSpec((FH, BB, 128), lambda i: (0, i, 0)),
            pl.BlockSpec((BB, F), lambda i: (i, 0)),
            pl.BlockSpec((BB, NUM_DENSE), lambda i: (i, 0)),
            full(NUM_DENSE, 1),
            full(1, 1),
            full(F, 128, H1),
            full(1, H1),
            full(H1, H2),
            full(1, H2),
            full(H2, H3),
            full(1, H3),
            full(1, 1),
            full(H3, 1),
            full(1, 1),
        ],
        out_specs=pl.BlockSpec((BB, 1), lambda i: (i, 0)),
        out_shape=jax.ShapeDtypeStruct((B, 1), jnp.float32),
    )(xw3a, xw3b, qoff, dense, wl, bl, w1r, b1, w2, b2, w3, b3, woh, wot, bo)


def kernel(dense_features, sparse_indices, tables, Wl, bl, W1, b1, W2, b2, W3, b3, Wo, bo):
    idx = sparse_indices.astype(jnp.int32)
    tabT = jnp.transpose(tables, (0, 2, 1))

    qsel = idx // Q
    vr = idx - qsel * Q
    floc = (jnp.arange(FH, dtype=jnp.int32) * Q)[None, :]

    def half_idx(h):
        mh = vr[:, h * FH:(h + 1) * FH] + floc
        m3 = mh.T.reshape(NW, RPW // 128, 128)
        return jnp.pad(m3, ((0, 0), (0, 16 - RPW // 128), (0, 0)))

    g_a = _relayout(tabT, 0)
    rows_a = _sc_gather(g_a.reshape(FH * Q, 128), half_idx(0))
    g_b = _relayout(tabT, FH)
    rows_b = _sc_gather(g_b.reshape(FH * Q, 128), half_idx(1))
    xw3a = rows_a.reshape(FH, B, 128)
    xw3b = rows_b.reshape(FH, B, 128)
    w1r = jnp.broadcast_to(W1.reshape(F, 1, D, H1), (F, 4, D, H1))
    w1r = w1r.reshape(F, 128, H1)
    out = _mlp(
        xw3a,
        xw3b,
        qsel,
        dense_features,
        Wl,
        bl.reshape(1, 1),
        w1r,
        b1.reshape(1, H1),
        W2,
        b2.reshape(1, H2),
        W3,
        b3.reshape(1, H3),
        Wo[0:1, :],
        Wo[1:, :],
        bo.reshape(1, 1),
    )
    return out


# speedup vs baseline: 8.6169x; 1.0069x over previous
import functools

import jax
import jax.numpy as jnp
from jax import lax
from jax.experimental import pallas as pl
from jax.experimental.pallas import tpu as pltpu
from jax.experimental.pallas import tpu_sc as plsc

B = 4096
NUM_DENSE = 13
F = 26
V = 100000
D = 32
H1, H2, H3 = 1024, 512, 256

Q = 12544
BG = 12544
NQB = Q // BG

NC, NS = 2, 16
NW = NC * NS
FH = F // 2
N_ROWS_H = B * FH
RPW = N_ROWS_H // NW
CHUNK = 128
NCHK = RPW // CHUNK


def _pack_pair(lo, hi):
    lo16 = lax.bitcast_convert_type(lo.astype(jnp.bfloat16), jnp.uint16)
    hi16 = lax.bitcast_convert_type(hi.astype(jnp.bfloat16), jnp.uint16)
    word = lo16.astype(jnp.uint32) | (hi16.astype(jnp.uint32) << 16)
    return lax.bitcast_convert_type(word, jnp.float32)


def _relayout_body(t0, t1, t2, t3, t4, t5, t6, t7, out):
    ts = (t0, t1, t2, t3, t4, t5, t6, t7)
    packed = [_pack_pair(ts[2 * a][0], ts[2 * a + 1][0]) for a in range(4)]
    tstack = jnp.concatenate(packed, axis=0)
    out[0] = jnp.transpose(tstack)


def _relayout(tabT, f0):
    def in_spec(s):
        return pl.BlockSpec(
            (1, D, BG), lambda f, g, s=s: (f0 + f, 0, NQB * s + g))

    return pl.pallas_call(
        _relayout_body,
        grid=(FH, NQB),
        in_specs=[in_spec(s) for s in range(8)],
        out_specs=pl.BlockSpec((1, BG, 128), lambda f, g: (f, g, 0)),
        out_shape=jax.ShapeDtypeStruct((FH, Q, 128), jnp.float32),
    )(*([tabT] * 8))


def _sc_gather(g2, idx3d):
    mesh = plsc.VectorSubcoreMesh(core_axis_name="c", subcore_axis_name="s")

    @functools.partial(
        pl.kernel,
        out_type=jax.ShapeDtypeStruct((N_ROWS_H, 128), jnp.float32),
        mesh=mesh,
        scratch_types=[
            pltpu.VMEM((16, 128), jnp.int32),
            pltpu.VMEM((2, CHUNK, 128), jnp.float32),
            pltpu.SemaphoreType.DMA,
            pltpu.SemaphoreType.DMA,
        ],
        compiler_params=pltpu.CompilerParams(use_tc_tiling_on_sc=True),
    )
    def k(g_hbm, idx_hbm, out_hbm, idx_v, rows_v, gsem, wsem):
        wid = lax.axis_index("s") * NC + lax.axis_index("c")
        pltpu.sync_copy(idx_hbm.at[wid], idx_v)

        def fire(c):
            return pltpu.async_copy(
                g_hbm.at[idx_v.at[c]], rows_v.at[c % 2], gsem)

        writes = []
        gather = fire(0)
        base = wid * RPW
        for c in range(NCHK):
            gather.wait()
            writes.append(
                pltpu.async_copy(
                    rows_v.at[c % 2],
                    out_hbm.at[pl.ds(base + c * CHUNK, CHUNK)],
                    wsem,
                )
            )
            if c + 1 < NCHK:
                if len(writes) >= 2:
                    writes.pop(0).wait()
                gather = fire(c + 1)
        for w in writes:
            w.wait()

    return k(g2, idx3d)


def _mlp_body(xwa, xwb, qoff, dense, wl, bl, w1r, b1, w2, b2, w3, b3, woh,
              wot, bo, out):
    f32 = jnp.float32
    lane_q = lax.broadcasted_iota(jnp.int32, (1, 128), 1) // D
    himask = jnp.uint32(0xFFFF0000)
    h = jnp.broadcast_to(b1[:], (qoff.shape[0], H1))
    for f in range(F):
        xw = xwa if f < FH else xwb
        qv = qoff[:, f:f + 1]
        mask = (lane_q == (qv >> 1)).astype(f32)
        shift = ((1 - (qv & 1)) << 4).astype(jnp.uint32)
        xu = lax.bitcast_convert_type(xw[f % FH], jnp.uint32)
        xv = lax.bitcast_convert_type((xu << shift) & himask, f32)
        xm = xv * mask
        h = h + jnp.dot(xm, w1r[f], preferred_element_type=f32)
    h = jnp.maximum(h, 0.0)
    wide = jnp.dot(dense[:], wl[:], preferred_element_type=f32) + bl[0, 0]
    h = jnp.dot(h, w2[:], preferred_element_type=f32) + b2[:]
    h = jnp.maximum(h, 0.0)
    h = jnp.dot(h, w3[:], preferred_element_type=f32) + b3[:]
    h = jnp.maximum(h, 0.0)
    logit = wide * woh[0, 0] + jnp.dot(h, wot[:], preferred_element_type=f32)
    out[:] = jax.nn.sigmoid(logit + bo[0, 0])


def _mlp(xw3a, xw3b, qoff, dense, wl, bl, w1r, b1, w2, b2, w3, b3, woh, wot,
         bo):
    BB = 512
    grid = (B // BB,)

    def full(*dims):
        return pl.BlockSpec(dims, lambda i, dims=dims: tuple(0 for _ in dims))

    return pl.pallas_call(
        _mlp_body,
        grid=grid,
        in_specs=[
            pl.BlockSpec((FH, BB, 128), lambda i: (0, i, 0)),
            pl.BlockSpec((FH, BB, 128), lambda i: (0, i, 0)),
            pl.BlockSpec((BB, F), lambda i: (i, 0)),
            pl.BlockSpec((BB, NUM_DENSE), lambda i: (i, 0)),
            full(NUM_DENSE, 1),
            full(1, 1),
            full(F, 128, H1),
            full(1, H1),
            full(H1, H2),
            full(1, H2),
            full(H2, H3),
            full(1, H3),
            full(1, 1),
            full(H3, 1),
            full(1, 1),
        ],
        out_specs=pl.BlockSpec((BB, 1), lambda i: (i, 0)),
        out_shape=jax.ShapeDtypeStruct((B, 1), jnp.float32),
    )(xw3a, xw3b, qoff, dense, wl, bl, w1r, b1, w2, b2, w3, b3, woh, wot, bo)


def kernel(dense_features, sparse_indices, tables, Wl, bl, W1, b1, W2, b2, W3, b3, Wo, bo):
    idx = sparse_indices.astype(jnp.int32)
    tabT = jnp.transpose(tables, (0, 2, 1))

    qsel = idx // Q
    vr = idx - qsel * Q
    floc = (jnp.arange(FH, dtype=jnp.int32) * Q)[None, :]

    def half_idx(h):
        mh = vr[:, h * FH:(h + 1) * FH] + floc
        m3 = mh.T.reshape(NW, RPW // 128, 128)
        return jnp.pad(m3, ((0, 0), (0, 16 - RPW // 128), (0, 0)))

    g_a = _relayout(tabT, 0)
    rows_a = _sc_gather(g_a.reshape(FH * Q, 128), half_idx(0))
    g_b = _relayout(tabT, FH)
    rows_b = _sc_gather(g_b.reshape(FH * Q, 128), half_idx(1))
    xw3a = rows_a.reshape(FH, B, 128)
    xw3b = rows_b.reshape(FH, B, 128)
    w1r = jnp.broadcast_to(W1.reshape(F, 1, D, H1), (F, 4, D, H1))
    w1r = w1r.reshape(F, 128, H1)
    out = _mlp(
        xw3a,
        xw3b,
        qsel,
        dense_features,
        Wl,
        bl.reshape(1, 1),
        w1r,
        b1.reshape(1, H1),
        W2,
        b2.reshape(1, H2),
        W3,
        b3.reshape(1, H3),
        Wo[0:1, :],
        Wo[1:, :],
        bo.reshape(1, 1),
    )
    return out


# speedup vs baseline: 8.7865x; 1.0197x over previous
import functools

import jax
import jax.numpy as jnp
from jax import lax
from jax.experimental import pallas as pl
from jax.experimental.pallas import tpu as pltpu
from jax.experimental.pallas import tpu_sc as plsc

B = 4096
NUM_DENSE = 13
F = 26
V = 100000
D = 32
H1, H2, H3 = 1024, 512, 256

Q = 12544
BG = 12544
NQB = Q // BG

NC, NS = 2, 16
NW = NC * NS
FH = F // 2
N_ROWS_H = B * FH
RPW = N_ROWS_H // NW
CHUNK = 128
NCHK = RPW // CHUNK


def _pack_pair(lo, hi):
    lo16 = lax.bitcast_convert_type(lo.astype(jnp.bfloat16), jnp.uint16)
    hi16 = lax.bitcast_convert_type(hi.astype(jnp.bfloat16), jnp.uint16)
    word = lo16.astype(jnp.uint32) | (hi16.astype(jnp.uint32) << 16)
    return lax.bitcast_convert_type(word, jnp.float32)


def _relayout_body(t0, t1, t2, t3, t4, t5, t6, t7, out):
    ts = (t0, t1, t2, t3, t4, t5, t6, t7)
    packed = [_pack_pair(ts[2 * a][0], ts[2 * a + 1][0]) for a in range(4)]
    tstack = jnp.concatenate(packed, axis=0)
    out[0] = jnp.transpose(tstack)


def _relayout(tabT, f0):
    def in_spec(s):
        return pl.BlockSpec(
            (1, D, BG), lambda f, g, s=s: (f0 + f, 0, NQB * s + g))

    return pl.pallas_call(
        _relayout_body,
        grid=(FH, NQB),
        in_specs=[in_spec(s) for s in range(8)],
        out_specs=pl.BlockSpec((1, BG, 128), lambda f, g: (f, g, 0)),
        out_shape=jax.ShapeDtypeStruct((FH, Q, 128), jnp.float32),
    )(*([tabT] * 8))


def _sc_gather(g2, idx3d):
    mesh = plsc.VectorSubcoreMesh(core_axis_name="c", subcore_axis_name="s")

    @functools.partial(
        pl.kernel,
        out_type=jax.ShapeDtypeStruct((N_ROWS_H, 128), jnp.float32),
        mesh=mesh,
        scratch_types=[
            pltpu.VMEM((16, 128), jnp.int32),
            pltpu.VMEM((3, CHUNK, 128), jnp.float32),
            pltpu.SemaphoreType.DMA,
            pltpu.SemaphoreType.DMA,
        ],
        compiler_params=pltpu.CompilerParams(use_tc_tiling_on_sc=True),
    )
    def k(g_hbm, idx_hbm, out_hbm, idx_v, rows_v, gsem, wsem):
        wid = lax.axis_index("s") * NC + lax.axis_index("c")
        pltpu.sync_copy(idx_hbm.at[wid], idx_v)

        def fire(c):
            return pltpu.async_copy(
                g_hbm.at[idx_v.at[c]], rows_v.at[c % 3], gsem)

        writes = []
        gathers = [fire(0), fire(1)]
        base = wid * RPW
        for c in range(NCHK):
            gathers.pop(0).wait()
            writes.append(
                pltpu.async_copy(
                    rows_v.at[c % 3],
                    out_hbm.at[pl.ds(base + c * CHUNK, CHUNK)],
                    wsem,
                )
            )
            if c + 2 < NCHK:
                if len(writes) >= 3:
                    writes.pop(0).wait()
                gathers.append(fire(c + 2))
        for w in writes:
            w.wait()

    return k(g2, idx3d)


def _mlp_body(xwa, xwb, qoff, dense, wl, bl, w1r, b1, w2, b2, w3, b3, woh,
              wot, bo, out):
    f32 = jnp.float32
    lane_q = lax.broadcasted_iota(jnp.int32, (1, 128), 1) // D
    himask = jnp.uint32(0xFFFF0000)
    h = jnp.broadcast_to(b1[:], (qoff.shape[0], H1))
    for f in range(F):
        xw = xwa if f < FH else xwb
        qv = qoff[:, f:f + 1]
        mask = (lane_q == (qv >> 1)).astype(f32)
        shift = ((1 - (qv & 1)) << 4).astype(jnp.uint32)
        xu = lax.bitcast_convert_type(xw[f % FH], jnp.uint32)
        xv = lax.bitcast_convert_type((xu << shift) & himask, f32)
        xm = xv * mask
        h = h + jnp.dot(xm, w1r[f], preferred_element_type=f32)
    h = jnp.maximum(h, 0.0)
    wide = jnp.dot(dense[:], wl[:], preferred_element_type=f32) + bl[0, 0]
    h = jnp.dot(h, w2[:], preferred_element_type=f32) + b2[:]
    h = jnp.maximum(h, 0.0)
    h = jnp.dot(h, w3[:], preferred_element_type=f32) + b3[:]
    h = jnp.maximum(h, 0.0)
    logit = wide * woh[0, 0] + jnp.dot(h, wot[:], preferred_element_type=f32)
    out[:] = jax.nn.sigmoid(logit + bo[0, 0])


def _mlp(xw3a, xw3b, qoff, dense, wl, bl, w1r, b1, w2, b2, w3, b3, woh, wot,
         bo):
    BB = 512
    grid = (B // BB,)

    def full(*dims):
        return pl.BlockSpec(dims, lambda i, dims=dims: tuple(0 for _ in dims))

    return pl.pallas_call(
        _mlp_body,
        grid=grid,
        in_specs=[
            pl.BlockSpec((FH, BB, 128), lambda i: (0, i, 0)),
            pl.BlockSpec((FH, BB, 128), lambda i: (0, i, 0)),
            pl.BlockSpec((BB, F), lambda i: (i, 0)),
            pl.BlockSpec((BB, NUM_DENSE), lambda i: (i, 0)),
            full(NUM_DENSE, 1),
            full(1, 1),
            full(F, 128, H1),
            full(1, H1),
            full(H1, H2),
            full(1, H2),
            full(H2, H3),
            full(1, H3),
            full(1, 1),
            full(H3, 1),
            full(1, 1),
        ],
        out_specs=pl.BlockSpec((BB, 1), lambda i: (i, 0)),
        out_shape=jax.ShapeDtypeStruct((B, 1), jnp.float32),
    )(xw3a, xw3b, qoff, dense, wl, bl, w1r, b1, w2, b2, w3, b3, woh, wot, bo)


def kernel(dense_features, sparse_indices, tables, Wl, bl, W1, b1, W2, b2, W3, b3, Wo, bo):
    idx = sparse_indices.astype(jnp.int32)
    tabT = jnp.transpose(tables, (0, 2, 1))

    qsel = idx // Q
    vr = idx - qsel * Q
    floc = (jnp.arange(FH, dtype=jnp.int32) * Q)[None, :]

    def half_idx(h):
        mh = vr[:, h * FH:(h + 1) * FH] + floc
        m3 = mh.T.reshape(NW, RPW // 128, 128)
        return jnp.pad(m3, ((0, 0), (0, 16 - RPW // 128), (0, 0)))

    g_a = _relayout(tabT, 0)
    rows_a = _sc_gather(g_a.reshape(FH * Q, 128), half_idx(0))
    g_b = _relayout(tabT, FH)
    rows_b = _sc_gather(g_b.reshape(FH * Q, 128), half_idx(1))
    xw3a = rows_a.reshape(FH, B, 128)
    xw3b = rows_b.reshape(FH, B, 128)
    w1r = jnp.broadcast_to(W1.reshape(F, 1, D, H1), (F, 4, D, H1))
    w1r = w1r.reshape(F, 128, H1)
    out = _mlp(
        xw3a,
        xw3b,
        qsel,
        dense_features,
        Wl,
        bl.reshape(1, 1),
        w1r,
        b1.reshape(1, H1),
        W2,
        b2.reshape(1, H2),
        W3,
        b3.reshape(1, H3),
        Wo[0:1, :],
        Wo[1:, :],
        bo.reshape(1, 1),
    )
    return out
